# Initial kernel scaffold; baseline (speedup 1.0000x reference)
import jax
import jax.numpy as jnp
from jax.experimental import pallas as pl


def kernel(pred_logits, pred_boxes, target_sizes):
    raise NotImplementedError("write your pallas kernel here")


# speedup vs baseline: 8.7129x; 8.7129x over previous
import functools

import jax
import jax.numpy as jnp
from jax import lax
from jax.experimental import pallas as pl
from jax.experimental.pallas import tpu as pltpu
from jax.experimental.pallas import tpu_sc as plsc

_MAXDET = 100
_NMS_T = 0.5
_THRES = 0.001
_NEG = -1e30
_NP = 1024
_CP = 96
_ND = _NP // 16


def _sc_nms(logits_flat, boxes_flat, ts_pad, B, C):
    info = plsc.get_sparse_core_info()
    nc = info.num_cores
    mesh = plsc.VectorSubcoreMesh(core_axis_name="c", subcore_axis_name="s")

    @functools.partial(
        pl.kernel,
        out_type=[
            jax.ShapeDtypeStruct((B, 128), jnp.float32),
            jax.ShapeDtypeStruct((B, 128), jnp.int32),
            jax.ShapeDtypeStruct((B, 512), jnp.float32),
            jax.ShapeDtypeStruct((B, 128), jnp.int32),
        ],
        mesh=mesh,
        scratch_types=[
            pltpu.VMEM((C * _NP,), jnp.float32),
            pltpu.VMEM((5 * _NP + 16,), jnp.float32),
            pltpu.VMEM((4 * _NP,), jnp.float32),
            pltpu.VMEM((_CP,), jnp.float32),
            pltpu.VMEM((_CP,), jnp.int32),
            pltpu.VMEM((128,), jnp.float32),
            pltpu.VMEM((128,), jnp.int32),
            pltpu.VMEM((512,), jnp.float32),
            pltpu.VMEM((128,), jnp.int32),
            pltpu.VMEM((16,), jnp.int32),
            pltpu.VMEM((32,), jnp.float32),
            pltpu.VMEM((32,), jnp.int32),
        ],
    )
    def k(logits_hbm, boxes_hbm, ts_hbm, osc_hbm, olb_hbm, obx_hbm, ovd_hbm,
          s_v, geom_v, bx_v, rmax_v, rarg_v, osc_v, olb_v, obx_v, ovd_v, ts_v,
          red_f, red_i):
        wid = lax.axis_index("s") * nc + lax.axis_index("c")

        @pl.when(wid < B)
        def _():
            img = wid
            pltpu.sync_copy(logits_hbm.at[img], s_v)
            pltpu.sync_copy(boxes_hbm.at[img], bx_v)
            pltpu.sync_copy(ts_hbm.at[img], ts_v)

            iota = lax.iota(jnp.int32, 16)
            neg16 = jnp.full((16,), _NEG, jnp.float32)
            zero16i = jnp.zeros((16,), jnp.int32)
            bigi = jnp.full((16,), 1 << 30, jnp.int32)

            def blend_store(ref, idx, val):
                base = (idx // 16) * 16
                lane = idx - base
                old = ref[pl.ds(base, 16)]
                ref[pl.ds(base, 16)] = jnp.where(iota == lane, val, old)

            red_f[pl.ds(16, 16)] = neg16
            red_i[pl.ds(16, 16)] = bigi

            def hmax16(v):
                red_f[pl.ds(0, 16)] = v
                for sh in (8, 4, 2, 1):
                    m = jnp.maximum(red_f[pl.ds(0, 16)], red_f[pl.ds(sh, 16)])
                    red_f[pl.ds(0, 16)] = m
                return red_f[pl.ds(0, 16)][0]

            def hmin16i(v):
                red_i[pl.ds(0, 16)] = v
                for sh in (8, 4, 2, 1):
                    m = jnp.minimum(red_i[pl.ds(0, 16)], red_i[pl.ds(sh, 16)])
                    red_i[pl.ds(0, 16)] = m
                return red_i[pl.ds(0, 16)][0]

            def geom_body(kk, _):
                o = kk * 16
                cx = bx_v[pl.ds(o, 16)]
                cy = bx_v[pl.ds(_NP + o, 16)]
                w = bx_v[pl.ds(2 * _NP + o, 16)]
                h = bx_v[pl.ds(3 * _NP + o, 16)]
                x0 = cx - 0.5 * w
                y0 = cy - 0.5 * h
                x1 = cx + 0.5 * w
                y1 = cy + 0.5 * h
                geom_v[pl.ds(o, 16)] = x0
                geom_v[pl.ds(_NP + o, 16)] = y0
                geom_v[pl.ds(2 * _NP + o, 16)] = x1
                geom_v[pl.ds(3 * _NP + o, 16)] = y1
                geom_v[pl.ds(4 * _NP + o, 16)] = (x1 - x0) * (y1 - y0)
                return 0

            lax.fori_loop(0, _ND, geom_body, 0)

            def row_body(c, _):
                rb = c * _NP

                def slice_body(kk, carry):
                    vmax, varg = carry
                    o = rb + kk * 16
                    x = s_v[pl.ds(o, 16)]
                    p = 1.0 / (1.0 + jnp.exp(-x))
                    sv = jnp.where(p > _THRES, p, _NEG)
                    s_v[pl.ds(o, 16)] = sv
                    m = sv > vmax
                    nvec = iota + kk * 16
                    return jnp.where(m, sv, vmax), jnp.where(m, nvec, varg)

                vmax, varg = lax.fori_loop(0, _ND, slice_body, (neg16, zero16i))
                mrow = hmax16(vmax)
                nrow = hmin16i(jnp.where(vmax == mrow, varg, bigi))
                blend_store(rmax_v, c, mrow)
                blend_store(rarg_v, c, nrow)
                return 0

            def rpad_body(kk, _):
                rmax_v[pl.ds(kk * 16, 16)] = neg16
                rarg_v[pl.ds(kk * 16, 16)] = zero16i
                return 0

            lax.fori_loop(0, _CP // 16, rpad_body, 0)
            lax.fori_loop(0, C, row_body, 0)

            def zo_body(kk, _):
                o = kk * 16
                osc_v[pl.ds(o, 16)] = jnp.zeros((16,), jnp.float32)
                olb_v[pl.ds(o, 16)] = zero16i
                ovd_v[pl.ds(o, 16)] = zero16i
                return 0

            lax.fori_loop(0, 8, zo_body, 0)

            def zb_body(kk, _):
                obx_v[pl.ds(kk * 16, 16)] = jnp.zeros((16,), jnp.float32)
                return 0

            lax.fori_loop(0, 32, zb_body, 0)

            tsvec = ts_v[pl.ds(0, 16)]
            hf = tsvec[0].astype(jnp.float32)
            wf = tsvec[1].astype(jnp.float32)

            def it_body(i, _):
                def mx_body(kk, cur):
                    return jnp.maximum(cur, rmax_v[pl.ds(kk * 16, 16)])

                vm = lax.fori_loop(0, _CP // 16, mx_body, neg16)
                mglob = hmax16(vm)

                def key_body(kk, cur):
                    rm = rmax_v[pl.ds(kk * 16, 16)]
                    ra = rarg_v[pl.ds(kk * 16, 16)]
                    cvec = iota + kk * 16
                    return jnp.minimum(cur, jnp.where(rm == mglob, ra * 128 + cvec, bigi))

                j2 = hmin16i(lax.fori_loop(0, _CP // 16, key_body, bigi))
                ok = mglob > _THRES

                @pl.when(ok)
                def _():
                    n = j2 // 128
                    c = j2 - n * 128
                    x0b = geom_v[pl.ds(n, 16)][0]
                    y0b = geom_v[pl.ds(_NP + n, 16)][0]
                    x1b = geom_v[pl.ds(2 * _NP + n, 16)][0]
                    y1b = geom_v[pl.ds(3 * _NP + n, 16)][0]
                    areab = geom_v[pl.ds(4 * _NP + n, 16)][0]
                    rb = c * _NP

                    def upd_body(kk, carry):
                        vmax, varg = carry
                        o = kk * 16
                        x0 = geom_v[pl.ds(o, 16)]
                        y0 = geom_v[pl.ds(_NP + o, 16)]
                        x1 = geom_v[pl.ds(2 * _NP + o, 16)]
                        y1 = geom_v[pl.ds(3 * _NP + o, 16)]
                        ar = geom_v[pl.ds(4 * _NP + o, 16)]
                        sc = s_v[pl.ds(rb + o, 16)]
                        inter = jnp.maximum(jnp.minimum(x1, x1b) - jnp.maximum(x0, x0b), 0.0)
                        inter = inter * jnp.maximum(jnp.minimum(y1, y1b) - jnp.maximum(y0, y0b), 0.0)
                        iou = inter / jnp.maximum(areab + ar - inter, 1e-9)
                        nvec = iota + o
                        ns = jnp.where((iou > _NMS_T) | (nvec == n), _NEG, sc)
                        s_v[pl.ds(rb + o, 16)] = ns
                        m = ns > vmax
                        return jnp.where(m, ns, vmax), jnp.where(m, nvec, varg)

                    vmax, varg = lax.fori_loop(0, _ND, upd_body, (neg16, zero16i))
                    mrow = hmax16(vmax)
                    blend_store(rmax_v, c, mrow)
                    blend_store(rarg_v, c, hmin16i(jnp.where(vmax == mrow, varg, bigi)))
                    blend_store(osc_v, i, mglob)
                    blend_store(olb_v, i, c)
                    blend_store(ovd_v, i, jnp.int32(1))
                    pos = 4 * i
                    bbase = (pos // 16) * 16
                    l0 = pos - bbase
                    old = obx_v[pl.ds(bbase, 16)]
                    bv = jnp.where(iota == l0, x0b * wf, old)
                    bv = jnp.where(iota == l0 + 1, y0b * hf, bv)
                    bv = jnp.where(iota == l0 + 2, x1b * wf, bv)
                    bv = jnp.where(iota == l0 + 3, y1b * hf, bv)
                    obx_v[pl.ds(bbase, 16)] = bv

                return 0

            lax.fori_loop(0, _MAXDET, it_body, 0)

            pltpu.sync_copy(osc_v, osc_hbm.at[img])
            pltpu.sync_copy(olb_v, olb_hbm.at[img])
            pltpu.sync_copy(obx_v, obx_hbm.at[img])
            pltpu.sync_copy(ovd_v, ovd_hbm.at[img])

    return k(logits_flat, boxes_flat, ts_pad)


def kernel(pred_logits, pred_boxes, target_sizes):
    B, N, C = pred_logits.shape
    lt = jnp.transpose(pred_logits, (0, 2, 1))
    lt = jnp.pad(lt, ((0, 0), (0, 0), (0, _NP - N)), constant_values=-1e9)
    lflat = lt.reshape(B, C * _NP)
    bt = jnp.transpose(pred_boxes, (0, 2, 1))
    bt = jnp.pad(bt, ((0, 0), (0, 0), (0, _NP - N)))
    bflat = bt.reshape(B, 4 * _NP)
    tsp = jnp.pad(target_sizes, ((0, 0), (0, 16 - target_sizes.shape[1])))
    osc, olb, obx, ovd = _sc_nms(lflat, bflat, tsp, B, C)
    return (
        osc[:, :_MAXDET],
        olb[:, :_MAXDET],
        obx.reshape(B, 128, 4)[:, :_MAXDET, :],
        ovd[:, :_MAXDET] != 0,
    )


# speedup vs baseline: 8.7331x; 1.0023x over previous
import functools

import jax
import jax.numpy as jnp
from jax import lax
from jax.experimental import pallas as pl
from jax.experimental.pallas import tpu as pltpu
from jax.experimental.pallas import tpu_sc as plsc

_MAXDET = 100
_NMS_T = 0.5
_THRES = 0.001
_NEG = -1e30
_NP = 1024
_CP = 96
_ND = _NP // 16


def _sc_nms(logits_flat, boxes_flat, ts_pad, B, C):
    info = plsc.get_sparse_core_info()
    nc = info.num_cores
    mesh = plsc.VectorSubcoreMesh(core_axis_name="c", subcore_axis_name="s")

    @functools.partial(
        pl.kernel,
        out_type=[
            jax.ShapeDtypeStruct((B, 128), jnp.float32),
            jax.ShapeDtypeStruct((B, 128), jnp.int32),
            jax.ShapeDtypeStruct((B, 512), jnp.float32),
            jax.ShapeDtypeStruct((B, 128), jnp.int32),
        ],
        mesh=mesh,
        scratch_types=[
            pltpu.VMEM((C * _NP,), jnp.float32),
            pltpu.VMEM((5 * _NP + 16,), jnp.float32),
            pltpu.VMEM((4 * _NP,), jnp.float32),
            pltpu.VMEM((_CP,), jnp.float32),
            pltpu.VMEM((_CP,), jnp.int32),
            pltpu.VMEM((128,), jnp.float32),
            pltpu.VMEM((128,), jnp.int32),
            pltpu.VMEM((512,), jnp.float32),
            pltpu.VMEM((128,), jnp.int32),
            pltpu.VMEM((16,), jnp.int32),
            pltpu.VMEM((32,), jnp.float32),
            pltpu.VMEM((32,), jnp.int32),
        ],
    )
    def k(logits_hbm, boxes_hbm, ts_hbm, osc_hbm, olb_hbm, obx_hbm, ovd_hbm,
          s_v, geom_v, bx_v, rmax_v, rarg_v, osc_v, olb_v, obx_v, ovd_v, ts_v,
          red_f, red_i):
        wid = lax.axis_index("s") * nc + lax.axis_index("c")

        @pl.when(wid < B)
        def _():
            img = wid
            pltpu.sync_copy(logits_hbm.at[img], s_v)
            pltpu.sync_copy(boxes_hbm.at[img], bx_v)
            pltpu.sync_copy(ts_hbm.at[img], ts_v)

            iota = lax.iota(jnp.int32, 16)
            neg16 = jnp.full((16,), _NEG, jnp.float32)
            zero16i = jnp.zeros((16,), jnp.int32)
            bigi = jnp.full((16,), 1 << 30, jnp.int32)

            def blend_store(ref, idx, val):
                base = (idx // 16) * 16
                lane = idx - base
                old = ref[pl.ds(base, 16)]
                ref[pl.ds(base, 16)] = jnp.where(iota == lane, val, old)

            red_f[pl.ds(16, 16)] = neg16
            red_i[pl.ds(16, 16)] = bigi

            def hmax16(v):
                red_f[pl.ds(0, 16)] = v
                for sh in (8, 4, 2, 1):
                    m = jnp.maximum(red_f[pl.ds(0, 16)], red_f[pl.ds(sh, 16)])
                    red_f[pl.ds(0, 16)] = m
                return red_f[pl.ds(0, 16)][0]

            def hmin16i(v):
                red_i[pl.ds(0, 16)] = v
                for sh in (8, 4, 2, 1):
                    m = jnp.minimum(red_i[pl.ds(0, 16)], red_i[pl.ds(sh, 16)])
                    red_i[pl.ds(0, 16)] = m
                return red_i[pl.ds(0, 16)][0]

            @plsc.parallel_loop(0, _ND, unroll=4)
            def geom_body(kk):
                o = kk * 16
                cx = bx_v[pl.ds(o, 16)]
                cy = bx_v[pl.ds(_NP + o, 16)]
                w = bx_v[pl.ds(2 * _NP + o, 16)]
                h = bx_v[pl.ds(3 * _NP + o, 16)]
                x0 = cx - 0.5 * w
                y0 = cy - 0.5 * h
                x1 = cx + 0.5 * w
                y1 = cy + 0.5 * h
                geom_v[pl.ds(o, 16)] = x0
                geom_v[pl.ds(_NP + o, 16)] = y0
                geom_v[pl.ds(2 * _NP + o, 16)] = x1
                geom_v[pl.ds(3 * _NP + o, 16)] = y1
                geom_v[pl.ds(4 * _NP + o, 16)] = (x1 - x0) * (y1 - y0)

            def row_body(c, _):
                rb = c * _NP

                @plsc.parallel_loop(0, _ND, unroll=4, carry=(neg16, zero16i))
                def init_carry(kk, carry):
                    vmax, varg = carry
                    o = rb + kk * 16
                    x = s_v[pl.ds(o, 16)]
                    p = 1.0 / (1.0 + jnp.exp(-x))
                    sv = jnp.where(p > _THRES, p, _NEG)
                    s_v[pl.ds(o, 16)] = sv
                    m = sv > vmax
                    nvec = iota + kk * 16
                    return jnp.where(m, sv, vmax), jnp.where(m, nvec, varg)

                vmax, varg = init_carry
                mrow = hmax16(vmax)
                nrow = hmin16i(jnp.where(vmax == mrow, varg, bigi))
                blend_store(rmax_v, c, mrow)
                blend_store(rarg_v, c, nrow)
                return 0

            @plsc.parallel_loop(0, _CP // 16)
            def rpad_body(kk):
                rmax_v[pl.ds(kk * 16, 16)] = neg16
                rarg_v[pl.ds(kk * 16, 16)] = zero16i

            lax.fori_loop(0, C, row_body, 0)

            @plsc.parallel_loop(0, 8)
            def zo_body(kk):
                o = kk * 16
                osc_v[pl.ds(o, 16)] = jnp.zeros((16,), jnp.float32)
                olb_v[pl.ds(o, 16)] = zero16i
                ovd_v[pl.ds(o, 16)] = zero16i

            @plsc.parallel_loop(0, 32)
            def zb_body(kk):
                obx_v[pl.ds(kk * 16, 16)] = jnp.zeros((16,), jnp.float32)

            tsvec = ts_v[pl.ds(0, 16)]
            hf = tsvec[0].astype(jnp.float32)
            wf = tsvec[1].astype(jnp.float32)

            def it_body(i, _):
                @plsc.parallel_loop(0, _CP // 16, unroll=6, carry=neg16)
                def mx_body(kk, cur):
                    return jnp.maximum(cur, rmax_v[pl.ds(kk * 16, 16)])

                mglob = hmax16(mx_body)

                @plsc.parallel_loop(0, _CP // 16, unroll=6, carry=bigi)
                def key_body(kk, cur):
                    rm = rmax_v[pl.ds(kk * 16, 16)]
                    ra = rarg_v[pl.ds(kk * 16, 16)]
                    cvec = iota + kk * 16
                    return jnp.minimum(cur, jnp.where(rm == mglob, ra * 128 + cvec, bigi))

                j2 = hmin16i(key_body)
                ok = mglob > _THRES

                @pl.when(ok)
                def _():
                    n = j2 // 128
                    c = j2 - n * 128
                    x0b = geom_v[pl.ds(n, 16)][0]
                    y0b = geom_v[pl.ds(_NP + n, 16)][0]
                    x1b = geom_v[pl.ds(2 * _NP + n, 16)][0]
                    y1b = geom_v[pl.ds(3 * _NP + n, 16)][0]
                    areab = geom_v[pl.ds(4 * _NP + n, 16)][0]
                    rb = c * _NP

                    @plsc.parallel_loop(0, _ND, unroll=4, carry=(neg16, zero16i))
                    def upd_body(kk, carry):
                        vmax, varg = carry
                        o = kk * 16
                        x0 = geom_v[pl.ds(o, 16)]
                        y0 = geom_v[pl.ds(_NP + o, 16)]
                        x1 = geom_v[pl.ds(2 * _NP + o, 16)]
                        y1 = geom_v[pl.ds(3 * _NP + o, 16)]
                        ar = geom_v[pl.ds(4 * _NP + o, 16)]
                        sc = s_v[pl.ds(rb + o, 16)]
                        inter = jnp.maximum(jnp.minimum(x1, x1b) - jnp.maximum(x0, x0b), 0.0)
                        inter = inter * jnp.maximum(jnp.minimum(y1, y1b) - jnp.maximum(y0, y0b), 0.0)
                        iou = inter / jnp.maximum(areab + ar - inter, 1e-9)
                        nvec = iota + o
                        ns = jnp.where((iou > _NMS_T) | (nvec == n), _NEG, sc)
                        s_v[pl.ds(rb + o, 16)] = ns
                        m = ns > vmax
                        return jnp.where(m, ns, vmax), jnp.where(m, nvec, varg)

                    vmax, varg = upd_body
                    mrow = hmax16(vmax)
                    blend_store(rmax_v, c, mrow)
                    blend_store(rarg_v, c, hmin16i(jnp.where(vmax == mrow, varg, bigi)))
                    blend_store(osc_v, i, mglob)
                    blend_store(olb_v, i, c)
                    blend_store(ovd_v, i, jnp.int32(1))
                    pos = 4 * i
                    bbase = (pos // 16) * 16
                    l0 = pos - bbase
                    old = obx_v[pl.ds(bbase, 16)]
                    bv = jnp.where(iota == l0, x0b * wf, old)
                    bv = jnp.where(iota == l0 + 1, y0b * hf, bv)
                    bv = jnp.where(iota == l0 + 2, x1b * wf, bv)
                    bv = jnp.where(iota == l0 + 3, y1b * hf, bv)
                    obx_v[pl.ds(bbase, 16)] = bv

                return 0

            lax.fori_loop(0, _MAXDET, it_body, 0)

            pltpu.sync_copy(osc_v, osc_hbm.at[img])
            pltpu.sync_copy(olb_v, olb_hbm.at[img])
            pltpu.sync_copy(obx_v, obx_hbm.at[img])
            pltpu.sync_copy(ovd_v, ovd_hbm.at[img])

    return k(logits_flat, boxes_flat, ts_pad)


def kernel(pred_logits, pred_boxes, target_sizes):
    B, N, C = pred_logits.shape
    lt = jnp.transpose(pred_logits, (0, 2, 1))
    lt = jnp.pad(lt, ((0, 0), (0, 0), (0, _NP - N)), constant_values=-1e9)
    lflat = lt.reshape(B, C * _NP)
    bt = jnp.transpose(pred_boxes, (0, 2, 1))
    bt = jnp.pad(bt, ((0, 0), (0, 0), (0, _NP - N)))
    bflat = bt.reshape(B, 4 * _NP)
    tsp = jnp.pad(target_sizes, ((0, 0), (0, 16 - target_sizes.shape[1])))
    osc, olb, obx, ovd = _sc_nms(lflat, bflat, tsp, B, C)
    return (
        osc[:, :_MAXDET],
        olb[:, :_MAXDET],
        obx.reshape(B, 128, 4)[:, :_MAXDET, :],
        ovd[:, :_MAXDET] != 0,
    )


# speedup vs baseline: 8.8592x; 1.0144x over previous
import functools

import jax
import jax.numpy as jnp
from jax import lax
from jax.experimental import pallas as pl
from jax.experimental.pallas import tpu as pltpu
from jax.experimental.pallas import tpu_sc as plsc

_MAXDET = 100
_NMS_T = 0.5
_THRES = 0.001
_NEG = -1e30
_NP = 1024
_CP = 96
_ND = _NP // 16


def _sc_nms(logits_flat, boxes_flat, ts_pad, B, C):
    info = plsc.get_sparse_core_info()
    nc = info.num_cores
    mesh = plsc.VectorSubcoreMesh(core_axis_name="c", subcore_axis_name="s")

    @functools.partial(
        pl.kernel,
        out_type=[
            jax.ShapeDtypeStruct((B, 128), jnp.float32),
            jax.ShapeDtypeStruct((B, 128), jnp.int32),
            jax.ShapeDtypeStruct((B, 512), jnp.float32),
            jax.ShapeDtypeStruct((B, 128), jnp.int32),
        ],
        mesh=mesh,
        scratch_types=[
            pltpu.VMEM((C * _NP,), jnp.float32),
            pltpu.VMEM((4 * _NP + 16,), jnp.float32),
            pltpu.VMEM((4 * _NP,), jnp.float32),
            pltpu.VMEM((_CP,), jnp.float32),
            pltpu.VMEM((_CP,), jnp.int32),
            pltpu.VMEM((128,), jnp.float32),
            pltpu.VMEM((128,), jnp.int32),
            pltpu.VMEM((512,), jnp.float32),
            pltpu.VMEM((128,), jnp.int32),
            pltpu.VMEM((16,), jnp.int32),
            pltpu.VMEM((32,), jnp.float32),
            pltpu.VMEM((32,), jnp.int32),
        ],
    )
    def k(logits_hbm, boxes_hbm, ts_hbm, osc_hbm, olb_hbm, obx_hbm, ovd_hbm,
          s_v, geom_v, bx_v, rmax_v, rarg_v, osc_v, olb_v, obx_v, ovd_v, ts_v,
          red_f, red_i):
        wid = lax.axis_index("s") * nc + lax.axis_index("c")

        @pl.when(wid < B)
        def _():
            img = wid
            pltpu.sync_copy(logits_hbm.at[img], s_v)
            pltpu.sync_copy(boxes_hbm.at[img], bx_v)
            pltpu.sync_copy(ts_hbm.at[img], ts_v)

            iota = lax.iota(jnp.int32, 16)
            neg16 = jnp.full((16,), _NEG, jnp.float32)
            zero16i = jnp.zeros((16,), jnp.int32)
            bigi = jnp.full((16,), 1 << 30, jnp.int32)

            def blend_store(ref, idx, val):
                base = (idx // 16) * 16
                lane = idx - base
                old = ref[pl.ds(base, 16)]
                ref[pl.ds(base, 16)] = jnp.where(iota == lane, val, old)

            red_f[pl.ds(16, 16)] = neg16
            red_i[pl.ds(16, 16)] = bigi

            def hmax16(v):
                red_f[pl.ds(0, 16)] = v
                for sh in (8, 4, 2, 1):
                    m = jnp.maximum(red_f[pl.ds(0, 16)], red_f[pl.ds(sh, 16)])
                    red_f[pl.ds(0, 16)] = m
                return red_f[pl.ds(0, 16)][0]

            def hmin16i(v):
                red_i[pl.ds(0, 16)] = v
                for sh in (8, 4, 2, 1):
                    m = jnp.minimum(red_i[pl.ds(0, 16)], red_i[pl.ds(sh, 16)])
                    red_i[pl.ds(0, 16)] = m
                return red_i[pl.ds(0, 16)][0]

            @plsc.parallel_loop(0, _ND, unroll=4)
            def geom_body(kk):
                o = kk * 16
                cx = bx_v[pl.ds(o, 16)]
                cy = bx_v[pl.ds(_NP + o, 16)]
                w = bx_v[pl.ds(2 * _NP + o, 16)]
                h = bx_v[pl.ds(3 * _NP + o, 16)]
                x0 = cx - 0.5 * w
                y0 = cy - 0.5 * h
                x1 = cx + 0.5 * w
                y1 = cy + 0.5 * h
                geom_v[pl.ds(o, 16)] = x0
                geom_v[pl.ds(_NP + o, 16)] = y0
                geom_v[pl.ds(2 * _NP + o, 16)] = x1
                geom_v[pl.ds(3 * _NP + o, 16)] = y1

            def row_body(c, _):
                rb = c * _NP

                def init_one(kk):
                    o = rb + kk * 16
                    x = s_v[pl.ds(o, 16)]
                    p = 1.0 / (1.0 + jnp.exp(-x))
                    sv = jnp.where(p > _THRES, p, _NEG)
                    s_v[pl.ds(o, 16)] = sv
                    return sv, iota + kk * 16

                @plsc.parallel_loop(0, _ND // 4, carry=(neg16, zero16i))
                def init_carry(q, carry):
                    vmax, varg = carry
                    parts = [init_one(q * 4 + u) for u in range(4)]
                    for sv, nvec in parts:
                        m = sv > vmax
                        vmax = jnp.where(m, sv, vmax)
                        varg = jnp.where(m, nvec, varg)
                    return vmax, varg

                vmax, varg = init_carry
                mrow = hmax16(vmax)
                nrow = hmin16i(jnp.where(vmax == mrow, varg, bigi))
                blend_store(rmax_v, c, mrow)
                blend_store(rarg_v, c, nrow)
                return 0

            @plsc.parallel_loop(0, _CP // 16)
            def rpad_body(kk):
                rmax_v[pl.ds(kk * 16, 16)] = neg16
                rarg_v[pl.ds(kk * 16, 16)] = zero16i

            lax.fori_loop(0, C, row_body, 0)

            @plsc.parallel_loop(0, 8)
            def zo_body(kk):
                o = kk * 16
                osc_v[pl.ds(o, 16)] = jnp.zeros((16,), jnp.float32)
                olb_v[pl.ds(o, 16)] = zero16i
                ovd_v[pl.ds(o, 16)] = zero16i

            @plsc.parallel_loop(0, 32)
            def zb_body(kk):
                obx_v[pl.ds(kk * 16, 16)] = jnp.zeros((16,), jnp.float32)

            tsvec = ts_v[pl.ds(0, 16)]
            hf = tsvec[0].astype(jnp.float32)
            wf = tsvec[1].astype(jnp.float32)

            def it_body(i, _):
                @plsc.parallel_loop(0, _CP // 16, unroll=6, carry=neg16)
                def mx_body(kk, cur):
                    return jnp.maximum(cur, rmax_v[pl.ds(kk * 16, 16)])

                mglob = hmax16(mx_body)

                @plsc.parallel_loop(0, _CP // 16, unroll=6, carry=bigi)
                def key_body(kk, cur):
                    rm = rmax_v[pl.ds(kk * 16, 16)]
                    ra = rarg_v[pl.ds(kk * 16, 16)]
                    cvec = iota + kk * 16
                    return jnp.minimum(cur, jnp.where(rm == mglob, ra * 128 + cvec, bigi))

                j2 = hmin16i(key_body)
                ok = mglob > _THRES

                @pl.when(ok)
                def _():
                    n = j2 // 128
                    c = j2 - n * 128
                    x0b = geom_v[pl.ds(n, 16)][0]
                    y0b = geom_v[pl.ds(_NP + n, 16)][0]
                    x1b = geom_v[pl.ds(2 * _NP + n, 16)][0]
                    y1b = geom_v[pl.ds(3 * _NP + n, 16)][0]
                    areab = (x1b - x0b) * (y1b - y0b)
                    rb = c * _NP

                    def upd_one(kk):
                        o = kk * 16
                        x0 = geom_v[pl.ds(o, 16)]
                        y0 = geom_v[pl.ds(_NP + o, 16)]
                        x1 = geom_v[pl.ds(2 * _NP + o, 16)]
                        y1 = geom_v[pl.ds(3 * _NP + o, 16)]
                        ar = (x1 - x0) * (y1 - y0)
                        sc = s_v[pl.ds(rb + o, 16)]
                        inter = jnp.maximum(jnp.minimum(x1, x1b) - jnp.maximum(x0, x0b), 0.0)
                        inter = inter * jnp.maximum(jnp.minimum(y1, y1b) - jnp.maximum(y0, y0b), 0.0)
                        iou = inter / jnp.maximum(areab + ar - inter, 1e-9)
                        nvec = iota + o
                        ns = jnp.where((iou > _NMS_T) | (nvec == n), _NEG, sc)
                        s_v[pl.ds(rb + o, 16)] = ns
                        return ns, nvec

                    @plsc.parallel_loop(0, _ND // 4, carry=(neg16, zero16i))
                    def upd_body(q, carry):
                        vmax, varg = carry
                        parts = [upd_one(q * 4 + u) for u in range(4)]
                        for ns, nvec in parts:
                            m = ns > vmax
                            vmax = jnp.where(m, ns, vmax)
                            varg = jnp.where(m, nvec, varg)
                        return vmax, varg

                    vmax, varg = upd_body
                    mrow = hmax16(vmax)
                    blend_store(rmax_v, c, mrow)
                    blend_store(rarg_v, c, hmin16i(jnp.where(vmax == mrow, varg, bigi)))
                    blend_store(osc_v, i, mglob)
                    blend_store(olb_v, i, c)
                    blend_store(ovd_v, i, jnp.int32(1))
                    pos = 4 * i
                    bbase = (pos // 16) * 16
                    l0 = pos - bbase
                    old = obx_v[pl.ds(bbase, 16)]
                    bv = jnp.where(iota == l0, x0b * wf, old)
                    bv = jnp.where(iota == l0 + 1, y0b * hf, bv)
                    bv = jnp.where(iota == l0 + 2, x1b * wf, bv)
                    bv = jnp.where(iota == l0 + 3, y1b * hf, bv)
                    obx_v[pl.ds(bbase, 16)] = bv

                return 0

            lax.fori_loop(0, _MAXDET, it_body, 0)

            pltpu.sync_copy(osc_v, osc_hbm.at[img])
            pltpu.sync_copy(olb_v, olb_hbm.at[img])
            pltpu.sync_copy(obx_v, obx_hbm.at[img])
            pltpu.sync_copy(ovd_v, ovd_hbm.at[img])

    return k(logits_flat, boxes_flat, ts_pad)


def kernel(pred_logits, pred_boxes, target_sizes):
    B, N, C = pred_logits.shape
    lt = jnp.transpose(pred_logits, (0, 2, 1))
    lt = jnp.pad(lt, ((0, 0), (0, 0), (0, _NP - N)), constant_values=-1e9)
    lflat = lt.reshape(B, C * _NP)
    bt = jnp.transpose(pred_boxes, (0, 2, 1))
    bt = jnp.pad(bt, ((0, 0), (0, 0), (0, _NP - N)))
    bflat = bt.reshape(B, 4 * _NP)
    tsp = jnp.pad(target_sizes, ((0, 0), (0, 16 - target_sizes.shape[1])))
    osc, olb, obx, ovd = _sc_nms(lflat, bflat, tsp, B, C)
    return (
        osc[:, :_MAXDET],
        olb[:, :_MAXDET],
        obx.reshape(B, 128, 4)[:, :_MAXDET, :],
        ovd[:, :_MAXDET] != 0,
    )


# speedup vs baseline: 19.9117x; 2.2476x over previous
import functools

import jax
import jax.numpy as jnp
from jax import lax
from jax.experimental import pallas as pl
from jax.experimental.pallas import tpu as pltpu
from jax.experimental.pallas import tpu_sc as plsc

_MAXDET = 100
_NMS_T = 0.5
_THRES = 0.001
_NEG = -1e30
_NP = 1024
_CP = 96
_ND = _NP // 16


def _sc_nms(logits_flat, boxes_flat, ts_pad, B, C):
    info = plsc.get_sparse_core_info()
    nc = info.num_cores
    mesh = plsc.VectorSubcoreMesh(core_axis_name="c", subcore_axis_name="s")

    @functools.partial(
        pl.kernel,
        out_type=[
            jax.ShapeDtypeStruct((B, 128), jnp.float32),
            jax.ShapeDtypeStruct((B, 128), jnp.int32),
            jax.ShapeDtypeStruct((B, 512), jnp.float32),
            jax.ShapeDtypeStruct((B, 128), jnp.int32),
        ],
        mesh=mesh,
        scratch_types=[
            pltpu.VMEM((C * _NP,), jnp.float32),
            pltpu.VMEM((4 * _NP + 16,), jnp.float32),
            pltpu.VMEM((4 * _NP,), jnp.float32),
            pltpu.VMEM((_CP,), jnp.float32),
            pltpu.VMEM((_CP,), jnp.int32),
            pltpu.VMEM((128,), jnp.float32),
            pltpu.VMEM((128,), jnp.int32),
            pltpu.VMEM((512,), jnp.float32),
            pltpu.VMEM((128,), jnp.int32),
            pltpu.VMEM((16,), jnp.int32),
            pltpu.VMEM((32,), jnp.float32),
            pltpu.VMEM((32,), jnp.int32),
        ],
    )
    def k(logits_hbm, boxes_hbm, ts_hbm, osc_hbm, olb_hbm, obx_hbm, ovd_hbm,
          s_v, geom_v, bx_v, rmax_v, rarg_v, osc_v, olb_v, obx_v, ovd_v, ts_v,
          red_f, red_i):
        wid = lax.axis_index("s") * nc + lax.axis_index("c")

        @pl.when(wid < B)
        def _():
            img = wid
            pltpu.sync_copy(logits_hbm.at[img], s_v)
            pltpu.sync_copy(boxes_hbm.at[img], bx_v)
            pltpu.sync_copy(ts_hbm.at[img], ts_v)

            iota = lax.iota(jnp.int32, 16)
            neg16 = jnp.full((16,), _NEG, jnp.float32)
            zero16i = jnp.zeros((16,), jnp.int32)
            bigi = jnp.full((16,), 1 << 30, jnp.int32)

            def blend_store(ref, idx, val):
                base = (idx // 16) * 16
                lane = idx - base
                old = ref[pl.ds(base, 16)]
                ref[pl.ds(base, 16)] = jnp.where(iota == lane, val, old)

            red_f[pl.ds(16, 16)] = neg16
            red_i[pl.ds(16, 16)] = bigi

            def hmax16(v):
                red_f[pl.ds(0, 16)] = v
                for sh in (8, 4, 2, 1):
                    m = jnp.maximum(red_f[pl.ds(0, 16)], red_f[pl.ds(sh, 16)])
                    red_f[pl.ds(0, 16)] = m
                return red_f[pl.ds(0, 16)][0]

            def hmin16i(v):
                red_i[pl.ds(0, 16)] = v
                for sh in (8, 4, 2, 1):
                    m = jnp.minimum(red_i[pl.ds(0, 16)], red_i[pl.ds(sh, 16)])
                    red_i[pl.ds(0, 16)] = m
                return red_i[pl.ds(0, 16)][0]

            @plsc.parallel_loop(0, _ND, unroll=4)
            def geom_body(kk):
                o = kk * 16
                cx = bx_v[pl.ds(o, 16)]
                cy = bx_v[pl.ds(_NP + o, 16)]
                w = bx_v[pl.ds(2 * _NP + o, 16)]
                h = bx_v[pl.ds(3 * _NP + o, 16)]
                x0 = cx - 0.5 * w
                y0 = cy - 0.5 * h
                x1 = cx + 0.5 * w
                y1 = cy + 0.5 * h
                geom_v[pl.ds(o, 16)] = x0
                geom_v[pl.ds(_NP + o, 16)] = y0
                geom_v[pl.ds(2 * _NP + o, 16)] = x1
                geom_v[pl.ds(3 * _NP + o, 16)] = y1

            def row_body(c, _):
                rb = c * _NP

                @plsc.parallel_loop(0, _ND // 4, carry=(neg16, zero16i))
                def init_carry(q, carry):
                    vmax, varg = carry
                    parts = []
                    for u in range(4):
                        kk = q * 4 + u
                        x = s_v[pl.ds(rb + kk * 16, 16)]
                        p = 1.0 / (1.0 + jnp.exp(-x))
                        sv = jnp.where(p > _THRES, p, _NEG)
                        parts.append((kk, sv, iota + kk * 16))
                    for kk, sv, _ in parts:
                        s_v[pl.ds(rb + kk * 16, 16)] = sv
                    for _, sv, nvec in parts:
                        m = sv > vmax
                        vmax = jnp.where(m, sv, vmax)
                        varg = jnp.where(m, nvec, varg)
                    return vmax, varg

                vmax, varg = init_carry
                mrow = hmax16(vmax)
                nrow = hmin16i(jnp.where(vmax == mrow, varg, bigi))
                blend_store(rmax_v, c, mrow)
                blend_store(rarg_v, c, nrow)
                return 0

            @plsc.parallel_loop(0, _CP // 16)
            def rpad_body(kk):
                rmax_v[pl.ds(kk * 16, 16)] = neg16
                rarg_v[pl.ds(kk * 16, 16)] = zero16i

            lax.fori_loop(0, C, row_body, 0)

            @plsc.parallel_loop(0, 8)
            def zo_body(kk):
                o = kk * 16
                osc_v[pl.ds(o, 16)] = jnp.zeros((16,), jnp.float32)
                olb_v[pl.ds(o, 16)] = zero16i
                ovd_v[pl.ds(o, 16)] = zero16i

            @plsc.parallel_loop(0, 32)
            def zb_body(kk):
                obx_v[pl.ds(kk * 16, 16)] = jnp.zeros((16,), jnp.float32)

            tsvec = ts_v[pl.ds(0, 16)]
            hf = tsvec[0].astype(jnp.float32)
            wf = tsvec[1].astype(jnp.float32)

            def it_body(i, _):
                @plsc.parallel_loop(0, _CP // 16, unroll=6, carry=neg16)
                def mx_body(kk, cur):
                    return jnp.maximum(cur, rmax_v[pl.ds(kk * 16, 16)])

                mglob = hmax16(mx_body)

                @plsc.parallel_loop(0, _CP // 16, unroll=6, carry=bigi)
                def key_body(kk, cur):
                    rm = rmax_v[pl.ds(kk * 16, 16)]
                    ra = rarg_v[pl.ds(kk * 16, 16)]
                    cvec = iota + kk * 16
                    return jnp.minimum(cur, jnp.where(rm == mglob, ra * 128 + cvec, bigi))

                j2 = hmin16i(key_body)
                ok = mglob > _THRES

                @pl.when(ok)
                def _():
                    n = j2 // 128
                    c = j2 - n * 128
                    x0b = geom_v[pl.ds(n, 16)][0]
                    y0b = geom_v[pl.ds(_NP + n, 16)][0]
                    x1b = geom_v[pl.ds(2 * _NP + n, 16)][0]
                    y1b = geom_v[pl.ds(3 * _NP + n, 16)][0]
                    areab = (x1b - x0b) * (y1b - y0b)
                    rb = c * _NP

                    @plsc.parallel_loop(0, _ND // 4, carry=(neg16, zero16i))
                    def upd_body(q, carry):
                        vmax, varg = carry
                        parts = []
                        for u in range(4):
                            kk = q * 4 + u
                            o = kk * 16
                            x0 = geom_v[pl.ds(o, 16)]
                            y0 = geom_v[pl.ds(_NP + o, 16)]
                            x1 = geom_v[pl.ds(2 * _NP + o, 16)]
                            y1 = geom_v[pl.ds(3 * _NP + o, 16)]
                            ar = (x1 - x0) * (y1 - y0)
                            sc = s_v[pl.ds(rb + o, 16)]
                            inter = jnp.maximum(jnp.minimum(x1, x1b) - jnp.maximum(x0, x0b), 0.0)
                            inter = inter * jnp.maximum(jnp.minimum(y1, y1b) - jnp.maximum(y0, y0b), 0.0)
                            iou = inter / jnp.maximum(areab + ar - inter, 1e-9)
                            nvec = iota + o
                            ns = jnp.where((iou > _NMS_T) | (nvec == n), _NEG, sc)
                            parts.append((kk, ns, nvec))
                        for kk, ns, _ in parts:
                            s_v[pl.ds(rb + kk * 16, 16)] = ns
                        for _, ns, nvec in parts:
                            m = ns > vmax
                            vmax = jnp.where(m, ns, vmax)
                            varg = jnp.where(m, nvec, varg)
                        return vmax, varg

                    vmax, varg = upd_body
                    mrow = hmax16(vmax)
                    blend_store(rmax_v, c, mrow)
                    blend_store(rarg_v, c, hmin16i(jnp.where(vmax == mrow, varg, bigi)))
                    blend_store(osc_v, i, mglob)
                    blend_store(olb_v, i, c)
                    blend_store(ovd_v, i, jnp.int32(1))
                    pos = 4 * i
                    bbase = (pos // 16) * 16
                    l0 = pos - bbase
                    old = obx_v[pl.ds(bbase, 16)]
                    bv = jnp.where(iota == l0, x0b * wf, old)
                    bv = jnp.where(iota == l0 + 1, y0b * hf, bv)
                    bv = jnp.where(iota == l0 + 2, x1b * wf, bv)
                    bv = jnp.where(iota == l0 + 3, y1b * hf, bv)
                    obx_v[pl.ds(bbase, 16)] = bv

                return 0

            lax.fori_loop(0, _MAXDET, it_body, 0)

            pltpu.sync_copy(osc_v, osc_hbm.at[img])
            pltpu.sync_copy(olb_v, olb_hbm.at[img])
            pltpu.sync_copy(obx_v, obx_hbm.at[img])
            pltpu.sync_copy(ovd_v, ovd_hbm.at[img])

    return k(logits_flat, boxes_flat, ts_pad)


def kernel(pred_logits, pred_boxes, target_sizes):
    B, N, C = pred_logits.shape
    lt = jnp.transpose(pred_logits, (0, 2, 1))
    lt = jnp.pad(lt, ((0, 0), (0, 0), (0, _NP - N)), constant_values=-1e9)
    lflat = lt.reshape(B, C * _NP)
    bt = jnp.transpose(pred_boxes, (0, 2, 1))
    bt = jnp.pad(bt, ((0, 0), (0, 0), (0, _NP - N)))
    bflat = bt.reshape(B, 4 * _NP)
    tsp = jnp.pad(target_sizes, ((0, 0), (0, 16 - target_sizes.shape[1])))
    osc, olb, obx, ovd = _sc_nms(lflat, bflat, tsp, B, C)
    return (
        osc[:, :_MAXDET],
        olb[:, :_MAXDET],
        obx.reshape(B, 128, 4)[:, :_MAXDET, :],
        ovd[:, :_MAXDET] != 0,
    )


# speedup vs baseline: 24.8005x; 1.2455x over previous
import functools

import jax
import jax.numpy as jnp
from jax import lax
from jax.experimental import pallas as pl
from jax.experimental.pallas import tpu as pltpu
from jax.experimental.pallas import tpu_sc as plsc

_MAXDET = 100
_NMS_T = 0.5
_THRES = 0.001
_NEG = -1e30
_NP = 1024
_CP = 96
_ND = _NP // 16


def _sc_nms(logits_flat, boxes_flat, ts_pad, B, C):
    info = plsc.get_sparse_core_info()
    nc = info.num_cores
    mesh = plsc.VectorSubcoreMesh(core_axis_name="c", subcore_axis_name="s")

    @functools.partial(
        pl.kernel,
        out_type=[
            jax.ShapeDtypeStruct((B, 128), jnp.float32),
            jax.ShapeDtypeStruct((B, 128), jnp.int32),
            jax.ShapeDtypeStruct((B, 512), jnp.float32),
            jax.ShapeDtypeStruct((B, 128), jnp.int32),
        ],
        mesh=mesh,
        scratch_types=[
            pltpu.VMEM((C * _NP,), jnp.float32),
            pltpu.VMEM((4 * _NP + 16,), jnp.float32),
            pltpu.VMEM((4 * _NP,), jnp.float32),
            pltpu.VMEM((_CP,), jnp.float32),
            pltpu.VMEM((_CP,), jnp.int32),
            pltpu.VMEM((128,), jnp.float32),
            pltpu.VMEM((128,), jnp.int32),
            pltpu.VMEM((512,), jnp.float32),
            pltpu.VMEM((128,), jnp.int32),
            pltpu.VMEM((16,), jnp.int32),
            pltpu.VMEM((32,), jnp.float32),
            pltpu.VMEM((32,), jnp.int32),
        ],
    )
    def k(logits_hbm, boxes_hbm, ts_hbm, osc_hbm, olb_hbm, obx_hbm, ovd_hbm,
          s_v, geom_v, bx_v, rmax_v, rarg_v, osc_v, olb_v, obx_v, ovd_v, ts_v,
          red_f, red_i):
        wid = lax.axis_index("s") * nc + lax.axis_index("c")

        @pl.when(wid < B)
        def _():
            img = wid
            pltpu.sync_copy(logits_hbm.at[img], s_v)
            pltpu.sync_copy(boxes_hbm.at[img], bx_v)
            pltpu.sync_copy(ts_hbm.at[img], ts_v)

            iota = lax.iota(jnp.int32, 16)
            neg16 = jnp.full((16,), _NEG, jnp.float32)
            zero16i = jnp.zeros((16,), jnp.int32)
            bigi = jnp.full((16,), 1 << 30, jnp.int32)

            def blend_store(ref, idx, val):
                base = (idx // 16) * 16
                lane = idx - base
                old = ref[pl.ds(base, 16)]
                ref[pl.ds(base, 16)] = jnp.where(iota == lane, val, old)

            red_f[pl.ds(16, 16)] = neg16
            red_i[pl.ds(16, 16)] = bigi

            def hmax16(v):
                red_f[pl.ds(0, 16)] = v
                for sh in (8, 4, 2, 1):
                    m = jnp.maximum(red_f[pl.ds(0, 16)], red_f[pl.ds(sh, 16)])
                    red_f[pl.ds(0, 16)] = m
                return red_f[pl.ds(0, 16)][0]

            def hmin16i(v):
                red_i[pl.ds(0, 16)] = v
                for sh in (8, 4, 2, 1):
                    m = jnp.minimum(red_i[pl.ds(0, 16)], red_i[pl.ds(sh, 16)])
                    red_i[pl.ds(0, 16)] = m
                return red_i[pl.ds(0, 16)][0]

            @plsc.parallel_loop(0, _ND, unroll=4)
            def geom_body(kk):
                o = kk * 16
                cx = bx_v[pl.ds(o, 16)]
                cy = bx_v[pl.ds(_NP + o, 16)]
                w = bx_v[pl.ds(2 * _NP + o, 16)]
                h = bx_v[pl.ds(3 * _NP + o, 16)]
                x0 = cx - 0.5 * w
                y0 = cy - 0.5 * h
                x1 = cx + 0.5 * w
                y1 = cy + 0.5 * h
                geom_v[pl.ds(o, 16)] = x0
                geom_v[pl.ds(_NP + o, 16)] = y0
                geom_v[pl.ds(2 * _NP + o, 16)] = x1
                geom_v[pl.ds(3 * _NP + o, 16)] = y1

            def row_body(c, _):
                rb = c * _NP

                @plsc.parallel_loop(0, _ND // 8, carry=(neg16, zero16i))
                def init_carry(q, carry):
                    vmax, varg = carry
                    parts = []
                    for u in range(8):
                        kk = q * 8 + u
                        x = s_v[pl.ds(rb + kk * 16, 16)]
                        p = 1.0 / (1.0 + jnp.exp(-x))
                        sv = jnp.where(p > _THRES, p, _NEG)
                        parts.append((kk, sv, iota + kk * 16))
                    for kk, sv, _ in parts:
                        s_v[pl.ds(rb + kk * 16, 16)] = sv
                    for _, sv, nvec in parts:
                        m = sv > vmax
                        vmax = jnp.where(m, sv, vmax)
                        varg = jnp.where(m, nvec, varg)
                    return vmax, varg

                vmax, varg = init_carry
                mrow = hmax16(vmax)
                nrow = hmin16i(jnp.where(vmax == mrow, varg, bigi))
                blend_store(rmax_v, c, mrow)
                blend_store(rarg_v, c, nrow)
                return 0

            @plsc.parallel_loop(0, _CP // 16)
            def rpad_body(kk):
                rmax_v[pl.ds(kk * 16, 16)] = neg16
                rarg_v[pl.ds(kk * 16, 16)] = zero16i

            lax.fori_loop(0, C, row_body, 0)

            @plsc.parallel_loop(0, 8)
            def zo_body(kk):
                o = kk * 16
                osc_v[pl.ds(o, 16)] = jnp.zeros((16,), jnp.float32)
                olb_v[pl.ds(o, 16)] = zero16i
                ovd_v[pl.ds(o, 16)] = zero16i

            @plsc.parallel_loop(0, 32)
            def zb_body(kk):
                obx_v[pl.ds(kk * 16, 16)] = jnp.zeros((16,), jnp.float32)

            tsvec = ts_v[pl.ds(0, 16)]
            hf = tsvec[0].astype(jnp.float32)
            wf = tsvec[1].astype(jnp.float32)

            def it_body(i, _):
                @plsc.parallel_loop(0, _CP // 16, unroll=6, carry=neg16)
                def mx_body(kk, cur):
                    return jnp.maximum(cur, rmax_v[pl.ds(kk * 16, 16)])

                mglob = hmax16(mx_body)

                @plsc.parallel_loop(0, _CP // 16, unroll=6, carry=bigi)
                def key_body(kk, cur):
                    rm = rmax_v[pl.ds(kk * 16, 16)]
                    ra = rarg_v[pl.ds(kk * 16, 16)]
                    cvec = iota + kk * 16
                    return jnp.minimum(cur, jnp.where(rm == mglob, ra * 128 + cvec, bigi))

                j2 = hmin16i(key_body)
                ok = mglob > _THRES

                @pl.when(ok)
                def _():
                    n = j2 // 128
                    c = j2 - n * 128
                    x0b = geom_v[pl.ds(n, 16)][0]
                    y0b = geom_v[pl.ds(_NP + n, 16)][0]
                    x1b = geom_v[pl.ds(2 * _NP + n, 16)][0]
                    y1b = geom_v[pl.ds(3 * _NP + n, 16)][0]
                    areab = (x1b - x0b) * (y1b - y0b)
                    rb = c * _NP

                    @plsc.parallel_loop(0, _ND // 8, carry=(neg16, zero16i))
                    def upd_body(q, carry):
                        vmax, varg = carry
                        parts = []
                        for u in range(8):
                            kk = q * 8 + u
                            o = kk * 16
                            x0 = geom_v[pl.ds(o, 16)]
                            y0 = geom_v[pl.ds(_NP + o, 16)]
                            x1 = geom_v[pl.ds(2 * _NP + o, 16)]
                            y1 = geom_v[pl.ds(3 * _NP + o, 16)]
                            ar = (x1 - x0) * (y1 - y0)
                            sc = s_v[pl.ds(rb + o, 16)]
                            inter = jnp.maximum(jnp.minimum(x1, x1b) - jnp.maximum(x0, x0b), 0.0)
                            inter = inter * jnp.maximum(jnp.minimum(y1, y1b) - jnp.maximum(y0, y0b), 0.0)
                            iou = inter / jnp.maximum(areab + ar - inter, 1e-9)
                            nvec = iota + o
                            ns = jnp.where((iou > _NMS_T) | (nvec == n), _NEG, sc)
                            parts.append((kk, ns, nvec))
                        for kk, ns, _ in parts:
                            s_v[pl.ds(rb + kk * 16, 16)] = ns
                        for _, ns, nvec in parts:
                            m = ns > vmax
                            vmax = jnp.where(m, ns, vmax)
                            varg = jnp.where(m, nvec, varg)
                        return vmax, varg

                    vmax, varg = upd_body
                    mrow = hmax16(vmax)
                    blend_store(rmax_v, c, mrow)
                    blend_store(rarg_v, c, hmin16i(jnp.where(vmax == mrow, varg, bigi)))
                    blend_store(osc_v, i, mglob)
                    blend_store(olb_v, i, c)
                    blend_store(ovd_v, i, jnp.int32(1))
                    pos = 4 * i
                    bbase = (pos // 16) * 16
                    l0 = pos - bbase
                    old = obx_v[pl.ds(bbase, 16)]
                    bv = jnp.where(iota == l0, x0b * wf, old)
                    bv = jnp.where(iota == l0 + 1, y0b * hf, bv)
                    bv = jnp.where(iota == l0 + 2, x1b * wf, bv)
                    bv = jnp.where(iota == l0 + 3, y1b * hf, bv)
                    obx_v[pl.ds(bbase, 16)] = bv

                return 0

            lax.fori_loop(0, _MAXDET, it_body, 0)

            pltpu.sync_copy(osc_v, osc_hbm.at[img])
            pltpu.sync_copy(olb_v, olb_hbm.at[img])
            pltpu.sync_copy(obx_v, obx_hbm.at[img])
            pltpu.sync_copy(ovd_v, ovd_hbm.at[img])

    return k(logits_flat, boxes_flat, ts_pad)


def kernel(pred_logits, pred_boxes, target_sizes):
    B, N, C = pred_logits.shape
    lt = jnp.transpose(pred_logits, (0, 2, 1))
    lt = jnp.pad(lt, ((0, 0), (0, 0), (0, _NP - N)), constant_values=-1e9)
    lflat = lt.reshape(B, C * _NP)
    bt = jnp.transpose(pred_boxes, (0, 2, 1))
    bt = jnp.pad(bt, ((0, 0), (0, 0), (0, _NP - N)))
    bflat = bt.reshape(B, 4 * _NP)
    tsp = jnp.pad(target_sizes, ((0, 0), (0, 16 - target_sizes.shape[1])))
    osc, olb, obx, ovd = _sc_nms(lflat, bflat, tsp, B, C)
    return (
        osc[:, :_MAXDET],
        olb[:, :_MAXDET],
        obx.reshape(B, 128, 4)[:, :_MAXDET, :],
        ovd[:, :_MAXDET] != 0,
    )


# speedup vs baseline: 25.5016x; 1.0283x over previous
import functools

import jax
import jax.numpy as jnp
from jax import lax
from jax.experimental import pallas as pl
from jax.experimental.pallas import tpu as pltpu
from jax.experimental.pallas import tpu_sc as plsc

_MAXDET = 100
_NMS_T = 0.5
_THRES = 0.001
_NEG = -1e30
_NP = 1024
_CP = 96
_ND = _NP // 16


def _sc_nms(logits_flat, boxes_flat, ts_pad, B, C):
    info = plsc.get_sparse_core_info()
    nc = info.num_cores
    mesh = plsc.VectorSubcoreMesh(core_axis_name="c", subcore_axis_name="s")

    @functools.partial(
        pl.kernel,
        out_type=[
            jax.ShapeDtypeStruct((B, 128), jnp.float32),
            jax.ShapeDtypeStruct((B, 128), jnp.int32),
            jax.ShapeDtypeStruct((B, 512), jnp.float32),
            jax.ShapeDtypeStruct((B, 128), jnp.int32),
        ],
        mesh=mesh,
        scratch_types=[
            pltpu.VMEM((C * _NP,), jnp.float32),
            pltpu.VMEM((4 * _NP + 16,), jnp.float32),
            pltpu.VMEM((4 * _NP,), jnp.float32),
            pltpu.VMEM((_CP,), jnp.float32),
            pltpu.VMEM((_CP,), jnp.int32),
            pltpu.VMEM((128,), jnp.float32),
            pltpu.VMEM((128,), jnp.int32),
            pltpu.VMEM((512,), jnp.float32),
            pltpu.VMEM((128,), jnp.int32),
            pltpu.VMEM((16,), jnp.int32),
            pltpu.VMEM((32,), jnp.float32),
            pltpu.VMEM((32,), jnp.int32),
        ],
    )
    def k(logits_hbm, boxes_hbm, ts_hbm, osc_hbm, olb_hbm, obx_hbm, ovd_hbm,
          s_v, geom_v, bx_v, rmax_v, rarg_v, osc_v, olb_v, obx_v, ovd_v, ts_v,
          red_f, red_i):
        wid = lax.axis_index("s") * nc + lax.axis_index("c")

        @pl.when(wid < B)
        def _():
            img = wid
            pltpu.sync_copy(logits_hbm.at[img], s_v)
            pltpu.sync_copy(boxes_hbm.at[img], bx_v)
            pltpu.sync_copy(ts_hbm.at[img], ts_v)

            iota = lax.iota(jnp.int32, 16)
            neg16 = jnp.full((16,), _NEG, jnp.float32)
            zero16i = jnp.zeros((16,), jnp.int32)
            bigi = jnp.full((16,), 1 << 30, jnp.int32)

            def blend_store(ref, idx, val):
                base = (idx // 16) * 16
                lane = idx - base
                old = ref[pl.ds(base, 16)]
                ref[pl.ds(base, 16)] = jnp.where(iota == lane, val, old)

            red_f[pl.ds(16, 16)] = neg16
            red_i[pl.ds(16, 16)] = bigi

            def hargmax_pair(vals, keys):
                red_f[pl.ds(0, 16)] = vals
                red_i[pl.ds(0, 16)] = keys
                for sh in (8, 4, 2, 1):
                    a = red_f[pl.ds(0, 16)]
                    b = red_f[pl.ds(sh, 16)]
                    ka = red_i[pl.ds(0, 16)]
                    kb = red_i[pl.ds(sh, 16)]
                    gt = a > b
                    eq = a == b
                    red_f[pl.ds(0, 16)] = jnp.maximum(a, b)
                    red_i[pl.ds(0, 16)] = jnp.where(
                        gt, ka, jnp.where(eq, jnp.minimum(ka, kb), kb))
                return red_f[pl.ds(0, 16)][0], red_i[pl.ds(0, 16)][0]

            @plsc.parallel_loop(0, _ND, unroll=4)
            def geom_body(kk):
                o = kk * 16
                cx = bx_v[pl.ds(o, 16)]
                cy = bx_v[pl.ds(_NP + o, 16)]
                w = bx_v[pl.ds(2 * _NP + o, 16)]
                h = bx_v[pl.ds(3 * _NP + o, 16)]
                x0 = cx - 0.5 * w
                y0 = cy - 0.5 * h
                x1 = cx + 0.5 * w
                y1 = cy + 0.5 * h
                geom_v[pl.ds(o, 16)] = x0
                geom_v[pl.ds(_NP + o, 16)] = y0
                geom_v[pl.ds(2 * _NP + o, 16)] = x1
                geom_v[pl.ds(3 * _NP + o, 16)] = y1

            def row_body(c, _):
                rb = c * _NP

                @plsc.parallel_loop(0, _ND // 8, carry=(neg16, zero16i))
                def init_carry(q, carry):
                    vmax, varg = carry
                    parts = []
                    for u in range(8):
                        kk = q * 8 + u
                        x = s_v[pl.ds(rb + kk * 16, 16)]
                        sv = 1.0 / (1.0 + jnp.exp(-x))
                        parts.append((kk, sv, iota + kk * 16))
                    for kk, sv, _ in parts:
                        s_v[pl.ds(rb + kk * 16, 16)] = sv
                    for _, sv, nvec in parts:
                        m = sv > vmax
                        vmax = jnp.where(m, sv, vmax)
                        varg = jnp.where(m, nvec, varg)
                    return vmax, varg

                vmax, varg = init_carry
                mrow, nrow = hargmax_pair(vmax, varg)
                blend_store(rmax_v, c, mrow)
                blend_store(rarg_v, c, nrow)
                return 0

            @plsc.parallel_loop(0, _CP // 16)
            def rpad_body(kk):
                rmax_v[pl.ds(kk * 16, 16)] = neg16
                rarg_v[pl.ds(kk * 16, 16)] = zero16i

            lax.fori_loop(0, C, row_body, 0)

            @plsc.parallel_loop(0, 8)
            def zo_body(kk):
                o = kk * 16
                osc_v[pl.ds(o, 16)] = jnp.zeros((16,), jnp.float32)
                olb_v[pl.ds(o, 16)] = zero16i
                ovd_v[pl.ds(o, 16)] = zero16i

            @plsc.parallel_loop(0, 32)
            def zb_body(kk):
                obx_v[pl.ds(kk * 16, 16)] = jnp.zeros((16,), jnp.float32)

            tsvec = ts_v[pl.ds(0, 16)]
            hf = tsvec[0].astype(jnp.float32)
            wf = tsvec[1].astype(jnp.float32)

            def it_body(i, _):
                @plsc.parallel_loop(0, _CP // 16, carry=(neg16, bigi))
                def mx_body(kk, carry):
                    vmax, vkey = carry
                    rm = rmax_v[pl.ds(kk * 16, 16)]
                    ra = rarg_v[pl.ds(kk * 16, 16)]
                    key = ra * 128 + (iota + kk * 16)
                    gt = rm > vmax
                    eq = rm == vmax
                    nkey = jnp.where(gt, key, jnp.where(eq, jnp.minimum(vkey, key), vkey))
                    return jnp.maximum(rm, vmax), nkey

                mglob, j2 = hargmax_pair(*mx_body)
                ok = mglob > _THRES

                @pl.when(ok)
                def _():
                    n = j2 // 128
                    c = j2 - n * 128
                    x0b = geom_v[pl.ds(n, 16)][0]
                    y0b = geom_v[pl.ds(_NP + n, 16)][0]
                    x1b = geom_v[pl.ds(2 * _NP + n, 16)][0]
                    y1b = geom_v[pl.ds(3 * _NP + n, 16)][0]
                    areab = (x1b - x0b) * (y1b - y0b)
                    rb = c * _NP

                    @plsc.parallel_loop(0, _ND // 8, carry=(neg16, zero16i))
                    def upd_body(q, carry):
                        vmax, varg = carry
                        parts = []
                        for u in range(8):
                            kk = q * 8 + u
                            o = kk * 16
                            x0 = geom_v[pl.ds(o, 16)]
                            y0 = geom_v[pl.ds(_NP + o, 16)]
                            x1 = geom_v[pl.ds(2 * _NP + o, 16)]
                            y1 = geom_v[pl.ds(3 * _NP + o, 16)]
                            ar = (x1 - x0) * (y1 - y0)
                            sc = s_v[pl.ds(rb + o, 16)]
                            inter = jnp.maximum(jnp.minimum(x1, x1b) - jnp.maximum(x0, x0b), 0.0)
                            inter = inter * jnp.maximum(jnp.minimum(y1, y1b) - jnp.maximum(y0, y0b), 0.0)
                            iou = inter / jnp.maximum(areab + ar - inter, 1e-9)
                            nvec = iota + o
                            ns = jnp.where((iou > _NMS_T) | (nvec == n), _NEG, sc)
                            parts.append((kk, ns, nvec))
                        for kk, ns, _ in parts:
                            s_v[pl.ds(rb + kk * 16, 16)] = ns
                        for _, ns, nvec in parts:
                            m = ns > vmax
                            vmax = jnp.where(m, ns, vmax)
                            varg = jnp.where(m, nvec, varg)
                        return vmax, varg

                    vmax, varg = upd_body
                    mrow, nrow = hargmax_pair(vmax, varg)
                    blend_store(rmax_v, c, mrow)
                    blend_store(rarg_v, c, nrow)
                    blend_store(osc_v, i, mglob)
                    blend_store(olb_v, i, c)
                    blend_store(ovd_v, i, jnp.int32(1))
                    pos = 4 * i
                    bbase = (pos // 16) * 16
                    l0 = pos - bbase
                    old = obx_v[pl.ds(bbase, 16)]
                    bv = jnp.where(iota == l0, x0b * wf, old)
                    bv = jnp.where(iota == l0 + 1, y0b * hf, bv)
                    bv = jnp.where(iota == l0 + 2, x1b * wf, bv)
                    bv = jnp.where(iota == l0 + 3, y1b * hf, bv)
                    obx_v[pl.ds(bbase, 16)] = bv

                return 0

            lax.fori_loop(0, _MAXDET, it_body, 0)

            pltpu.sync_copy(osc_v, osc_hbm.at[img])
            pltpu.sync_copy(olb_v, olb_hbm.at[img])
            pltpu.sync_copy(obx_v, obx_hbm.at[img])
            pltpu.sync_copy(ovd_v, ovd_hbm.at[img])

    return k(logits_flat, boxes_flat, ts_pad)


def kernel(pred_logits, pred_boxes, target_sizes):
    B, N, C = pred_logits.shape
    lt = jnp.transpose(pred_logits, (0, 2, 1))
    lt = jnp.pad(lt, ((0, 0), (0, 0), (0, _NP - N)), constant_values=-1e9)
    lflat = lt.reshape(B, C * _NP)
    bt = jnp.transpose(pred_boxes, (0, 2, 1))
    bt = jnp.pad(bt, ((0, 0), (0, 0), (0, _NP - N)))
    bflat = bt.reshape(B, 4 * _NP)
    tsp = jnp.pad(target_sizes, ((0, 0), (0, 16 - target_sizes.shape[1])))
    osc, olb, obx, ovd = _sc_nms(lflat, bflat, tsp, B, C)
    return (
        osc[:, :_MAXDET],
        olb[:, :_MAXDET],
        obx.reshape(B, 128, 4)[:, :_MAXDET, :],
        ovd[:, :_MAXDET] != 0,
    )


# speedup vs baseline: 28.9101x; 1.1337x over previous
import functools

import jax
import jax.numpy as jnp
from jax import lax
from jax.experimental import pallas as pl
from jax.experimental.pallas import tpu as pltpu
from jax.experimental.pallas import tpu_sc as plsc

_MAXDET = 100
_NMS_T = 0.5
_THRES = 0.001
_NEG = -1e30
_NP = 1024
_CP = 96
_ND = _NP // 16
_SH = 24


def _sc_nms(logits_flat, boxes_flat, ts_pad, B):
    mesh = plsc.VectorSubcoreMesh(core_axis_name="c", subcore_axis_name="s")

    @functools.partial(
        pl.kernel,
        out_type=[
            jax.ShapeDtypeStruct((B, 128), jnp.float32),
            jax.ShapeDtypeStruct((B, 128), jnp.int32),
            jax.ShapeDtypeStruct((B, 512), jnp.float32),
            jax.ShapeDtypeStruct((B, 128), jnp.int32),
        ],
        mesh=mesh,
        scratch_types=[
            pltpu.VMEM((_CP * _NP,), jnp.float32),
            pltpu.VMEM((4 * _NP + 16,), jnp.float32),
            pltpu.VMEM((4 * _NP,), jnp.float32),
            pltpu.VMEM((_CP,), jnp.float32),
            pltpu.VMEM((_CP,), jnp.int32),
            pltpu.VMEM((128,), jnp.float32),
            pltpu.VMEM((128,), jnp.int32),
            pltpu.VMEM((512,), jnp.float32),
            pltpu.VMEM((128,), jnp.int32),
            pltpu.VMEM((16,), jnp.int32),
            pltpu.VMEM((32,), jnp.float32),
            pltpu.VMEM((32,), jnp.int32),
            pltpu.VMEM((32,), jnp.float32),
            pltpu.VMEM((32,), jnp.int32),
            pltpu.VMEM((128,), jnp.float32),
            pltpu.VMEM((128,), jnp.int32),
            pltpu.VMEM_SHARED((4, 3 * _SH * _NP), jnp.float32),
            pltpu.VMEM_SHARED((4, 128), jnp.float32),
            pltpu.VMEM_SHARED((4, 128), jnp.int32),
        ],
    )
    def k(logits_hbm, boxes_hbm, ts_hbm, osc_hbm, olb_hbm, obx_hbm, ovd_hbm,
          s_v, geom_v, bx_v, rmax_v, rarg_v, osc_v, olb_v, obx_v, ovd_v, ts_v,
          red_f, red_i, lrm_v, lra_v, tmpf, tmpi, slab, rm_sh, ra_sh):
        sidx = lax.axis_index("s")
        cidx = lax.axis_index("c")
        li = sidx // 4
        q = sidx - li * 4
        img = cidx * 4 + li
        sbase = q * _SH * _NP

        iota = lax.iota(jnp.int32, 16)
        neg16 = jnp.full((16,), _NEG, jnp.float32)
        zero16i = jnp.zeros((16,), jnp.int32)
        bigi = jnp.full((16,), 1 << 30, jnp.int32)

        def blend_store(ref, idx, val):
            base = (idx // 16) * 16
            lane = idx - base
            old = ref[pl.ds(base, 16)]
            ref[pl.ds(base, 16)] = jnp.where(iota == lane, val, old)

        red_f[pl.ds(16, 16)] = neg16
        red_i[pl.ds(16, 16)] = bigi

        def hargmax_pair(vals, keys):
            red_f[pl.ds(0, 16)] = vals
            red_i[pl.ds(0, 16)] = keys
            for sh in (8, 4, 2, 1):
                a = red_f[pl.ds(0, 16)]
                b = red_f[pl.ds(sh, 16)]
                ka = red_i[pl.ds(0, 16)]
                kb = red_i[pl.ds(sh, 16)]
                gt = a > b
                eq = a == b
                red_f[pl.ds(0, 16)] = jnp.maximum(a, b)
                red_i[pl.ds(0, 16)] = jnp.where(
                    gt, ka, jnp.where(eq, jnp.minimum(ka, kb), kb))
            return red_f[pl.ds(0, 16)][0], red_i[pl.ds(0, 16)][0]

        pltpu.sync_copy(logits_hbm.at[img, pl.ds(sbase, _SH * _NP)],
                        s_v.at[pl.ds(0, _SH * _NP)])

        def row_body(c, _):
            rb = c * _NP

            @plsc.parallel_loop(0, _ND // 8, carry=(neg16, zero16i))
            def init_carry(g, carry):
                vmax, varg = carry
                parts = []
                for u in range(8):
                    kk = g * 8 + u
                    x = s_v[pl.ds(rb + kk * 16, 16)]
                    sv = 1.0 / (1.0 + jnp.exp(-x))
                    parts.append((kk, sv, iota + kk * 16))
                for kk, sv, _ in parts:
                    s_v[pl.ds(rb + kk * 16, 16)] = sv
                for _, sv, nvec in parts:
                    m = sv > vmax
                    vmax = jnp.where(m, sv, vmax)
                    varg = jnp.where(m, nvec, varg)
                return vmax, varg

            mrow, nrow = hargmax_pair(*init_carry)
            blend_store(lrm_v, c, mrow)
            blend_store(lra_v, c, nrow)
            return 0

        lax.fori_loop(0, _SH, row_body, 0)

        @pl.when(q > 0)
        def _():
            pltpu.sync_copy(s_v.at[pl.ds(0, _SH * _NP)],
                            slab.at[li, pl.ds(sbase - _SH * _NP, _SH * _NP)])

        pltpu.sync_copy(lrm_v, rm_sh.at[li, pl.ds(q * 32, 32)])
        pltpu.sync_copy(lra_v, ra_sh.at[li, pl.ds(q * 32, 32)])

        plsc.subcore_barrier()

        @pl.when(q == 0)
        def _():
            pltpu.sync_copy(slab.at[li],
                            s_v.at[pl.ds(_SH * _NP, 3 * _SH * _NP)])
            pltpu.sync_copy(rm_sh.at[li], tmpf)
            pltpu.sync_copy(ra_sh.at[li], tmpi)
            pltpu.sync_copy(boxes_hbm.at[img], bx_v)
            pltpu.sync_copy(ts_hbm.at[img], ts_v)

            for j in range(_CP // 16):
                cv = iota + 16 * j
                mv = tmpf[pl.ds(16 * j, 16)]
                av = tmpi[pl.ds(16 * j, 16)]
                for qq in (1, 2, 3):
                    sel = cv >= qq * _SH
                    mv = jnp.where(sel, tmpf[pl.ds(8 * qq + 16 * j, 16)], mv)
                    av = jnp.where(sel, tmpi[pl.ds(8 * qq + 16 * j, 16)], av)
                rmax_v[pl.ds(16 * j, 16)] = mv
                rarg_v[pl.ds(16 * j, 16)] = av

            @plsc.parallel_loop(0, _ND, unroll=4)
            def geom_body(kk):
                o = kk * 16
                cx = bx_v[pl.ds(o, 16)]
                cy = bx_v[pl.ds(_NP + o, 16)]
                w = bx_v[pl.ds(2 * _NP + o, 16)]
                h = bx_v[pl.ds(3 * _NP + o, 16)]
                geom_v[pl.ds(o, 16)] = cx - 0.5 * w
                geom_v[pl.ds(_NP + o, 16)] = cy - 0.5 * h
                geom_v[pl.ds(2 * _NP + o, 16)] = cx + 0.5 * w
                geom_v[pl.ds(3 * _NP + o, 16)] = cy + 0.5 * h

            @plsc.parallel_loop(0, 8)
            def zo_body(kk):
                o = kk * 16
                osc_v[pl.ds(o, 16)] = jnp.zeros((16,), jnp.float32)
                olb_v[pl.ds(o, 16)] = zero16i
                ovd_v[pl.ds(o, 16)] = zero16i

            @plsc.parallel_loop(0, 32)
            def zb_body(kk):
                obx_v[pl.ds(kk * 16, 16)] = jnp.zeros((16,), jnp.float32)

            tsvec = ts_v[pl.ds(0, 16)]
            hf = tsvec[0].astype(jnp.float32)
            wf = tsvec[1].astype(jnp.float32)

            def it_body(i, _):
                @plsc.parallel_loop(0, _CP // 16, carry=(neg16, bigi))
                def mx_body(kk, carry):
                    vmax, vkey = carry
                    rm = rmax_v[pl.ds(kk * 16, 16)]
                    ra = rarg_v[pl.ds(kk * 16, 16)]
                    key = ra * 128 + (iota + kk * 16)
                    gt = rm > vmax
                    eq = rm == vmax
                    nkey = jnp.where(gt, key, jnp.where(eq, jnp.minimum(vkey, key), vkey))
                    return jnp.maximum(rm, vmax), nkey

                mglob, j2 = hargmax_pair(*mx_body)
                ok = mglob > _THRES

                @pl.when(ok)
                def _():
                    n = j2 // 128
                    c = j2 - n * 128
                    x0b = geom_v[pl.ds(n, 16)][0]
                    y0b = geom_v[pl.ds(_NP + n, 16)][0]
                    x1b = geom_v[pl.ds(2 * _NP + n, 16)][0]
                    y1b = geom_v[pl.ds(3 * _NP + n, 16)][0]
                    areab = (x1b - x0b) * (y1b - y0b)
                    rb = c * _NP

                    @plsc.parallel_loop(0, _ND // 8, carry=(neg16, zero16i))
                    def upd_body(g, carry):
                        vmax, varg = carry
                        parts = []
                        for u in range(8):
                            kk = g * 8 + u
                            o = kk * 16
                            x0 = geom_v[pl.ds(o, 16)]
                            y0 = geom_v[pl.ds(_NP + o, 16)]
                            x1 = geom_v[pl.ds(2 * _NP + o, 16)]
                            y1 = geom_v[pl.ds(3 * _NP + o, 16)]
                            ar = (x1 - x0) * (y1 - y0)
                            sc = s_v[pl.ds(rb + o, 16)]
                            inter = jnp.maximum(jnp.minimum(x1, x1b) - jnp.maximum(x0, x0b), 0.0)
                            inter = inter * jnp.maximum(jnp.minimum(y1, y1b) - jnp.maximum(y0, y0b), 0.0)
                            iou = inter / jnp.maximum(areab + ar - inter, 1e-9)
                            nvec = iota + o
                            ns = jnp.where((iou > _NMS_T) | (nvec == n), _NEG, sc)
                            parts.append((kk, ns, nvec))
                        for kk, ns, _ in parts:
                            s_v[pl.ds(rb + kk * 16, 16)] = ns
                        for _, ns, nvec in parts:
                            m = ns > vmax
                            vmax = jnp.where(m, ns, vmax)
                            varg = jnp.where(m, nvec, varg)
                        return vmax, varg

                    mrow, nrow = hargmax_pair(*upd_body)
                    blend_store(rmax_v, c, mrow)
                    blend_store(rarg_v, c, nrow)
                    blend_store(osc_v, i, mglob)
                    blend_store(olb_v, i, c)
                    blend_store(ovd_v, i, jnp.int32(1))
                    pos = 4 * i
                    bbase = (pos // 16) * 16
                    l0 = pos - bbase
                    old = obx_v[pl.ds(bbase, 16)]
                    bv = jnp.where(iota == l0, x0b * wf, old)
                    bv = jnp.where(iota == l0 + 1, y0b * hf, bv)
                    bv = jnp.where(iota == l0 + 2, x1b * wf, bv)
                    bv = jnp.where(iota == l0 + 3, y1b * hf, bv)
                    obx_v[pl.ds(bbase, 16)] = bv

                return 0

            lax.fori_loop(0, _MAXDET, it_body, 0)

            pltpu.sync_copy(osc_v, osc_hbm.at[img])
            pltpu.sync_copy(olb_v, olb_hbm.at[img])
            pltpu.sync_copy(obx_v, obx_hbm.at[img])
            pltpu.sync_copy(ovd_v, ovd_hbm.at[img])

    return k(logits_flat, boxes_flat, ts_pad)


def kernel(pred_logits, pred_boxes, target_sizes):
    B, N, C = pred_logits.shape
    lt = jnp.transpose(pred_logits, (0, 2, 1))
    lt = jnp.pad(lt, ((0, 0), (0, _CP - C), (0, _NP - N)),
                 constant_values=-1e9)
    lflat = lt.reshape(B, _CP * _NP)
    bt = jnp.transpose(pred_boxes, (0, 2, 1))
    bt = jnp.pad(bt, ((0, 0), (0, 0), (0, _NP - N)))
    bflat = bt.reshape(B, 4 * _NP)
    tsp = jnp.pad(target_sizes, ((0, 0), (0, 16 - target_sizes.shape[1])))
    osc, olb, obx, ovd = _sc_nms(lflat, bflat, tsp, B)
    return (
        osc[:, :_MAXDET],
        olb[:, :_MAXDET],
        obx.reshape(B, 128, 4)[:, :_MAXDET, :],
        ovd[:, :_MAXDET] != 0,
    )
